# baseline (speedup 1.0000x reference)
import functools
import math

import jax
import jax.numpy as jnp
from jax import lax
from jax.experimental import pallas as pl
from jax.experimental.pallas import tpu as pltpu
from jax.experimental.pallas import tpu_sc as plsc

D = 128
S = 200
SCALE = math.sqrt(D)
LANES = 16
G = 100
NBUF = 4


def _make_kernel(N):
    info = plsc.get_sparse_core_info()
    NC, NS = info.num_cores, info.num_subcores
    NW = NC * NS
    n_per_w = N // NW
    n_chunks = n_per_w // S
    n_g = n_per_w // G
    n_groups = n_chunks // NBUF

    mesh = plsc.VectorSubcoreMesh(core_axis_name="c", subcore_axis_name="s")

    @functools.partial(
        pl.kernel,
        mesh=mesh,
        out_type=jax.ShapeDtypeStruct((N, D), jnp.float32),
        scratch_types=[
            pltpu.VMEM((S, D // 2), jnp.int32),
        ]
        + [pltpu.VMEM((2, G), jnp.int32) for _ in range(NBUF)]
        + [pltpu.VMEM((S, D), jnp.float32) for _ in range(NBUF)]
        + [pltpu.SemaphoreType.DMA for _ in range(3 * NBUF + 1)],
    )
    def k(x_hbm, pe_hbm, table_hbm, out_hbm, pe_v, *slabs_bufs_sems):
        idxs = slabs_bufs_sems[:NBUF]
        bufs = slabs_bufs_sems[NBUF:2 * NBUF]
        gsems = slabs_bufs_sems[2 * NBUF:3 * NBUF]
        ssems = slabs_bufs_sems[3 * NBUF:4 * NBUF]
        isems = slabs_bufs_sems[4 * NBUF:5 * NBUF]
        pesem = slabs_bufs_sems[5 * NBUF]

        wid = lax.axis_index("s") * NC + lax.axis_index("c")
        base_w = wid * n_per_w

        def start_idx_load(j, b):
            pltpu.async_copy(
                x_hbm.at[wid, pl.ds(2 * j, 2)], idxs[b], isems[b]
            )

        def wait_idx(b):
            pltpu.make_async_copy(
                x_hbm.at[wid, pl.ds(0, 2)], idxs[b], isems[b]
            ).wait()

        def start_gather(j, b):
            for h in range(2):
                pltpu.async_copy(
                    table_hbm.at[idxs[b].at[h, pl.ds(0, G // 2)]],
                    bufs[b].at[pl.ds(h * G, G // 2)],
                    gsems[b],
                )
                pltpu.async_copy(
                    table_hbm.at[idxs[b].at[h, pl.ds(G // 2, G // 2)]],
                    bufs[b].at[pl.ds(h * G + G // 2, G // 2)],
                    gsems[b],
                )

        def wait_gather(b):
            pltpu.make_async_copy(
                table_hbm.at[pl.ds(0, S)], bufs[b], gsems[b]
            ).wait()

        def start_scatter(j, b):
            pltpu.async_copy(
                bufs[b].at[pl.ds(0, 104)],
                out_hbm.at[pl.ds(base_w + j * S, 104)], ssems[b]
            )
            pltpu.async_copy(
                bufs[b].at[pl.ds(104, 96)],
                out_hbm.at[pl.ds(base_w + j * S + 104, 96)], ssems[b]
            )

        def wait_scatter(b):
            pltpu.make_async_copy(
                bufs[b], out_hbm.at[pl.ds(0, S)], ssems[b]
            ).wait()

        mask_hi = jnp.int32(-65536)

        def compute(b):
            buf = bufs[b]

            def do_row(r):
                for g in range(D // 32):
                    w = pe_v[r, pl.ds(g * LANES, LANES)]
                    lo = lax.bitcast_convert_type(
                        lax.shift_left(w, 16), jnp.float32)
                    hi = lax.bitcast_convert_type(
                        lax.bitwise_and(w, mask_hi), jnp.float32)
                    sl0 = pl.ds(g * 32, LANES)
                    sl1 = pl.ds(g * 32 + LANES, LANES)
                    buf[r, sl0] = buf[r, sl0] * SCALE + lo
                    buf[r, sl1] = buf[r, sl1] * SCALE + hi

            def row_body(r2, rc):
                do_row(2 * r2)
                do_row(2 * r2 + 1)
                return rc

            lax.fori_loop(0, S // 2, row_body, 0)

        for b in range(NBUF):
            start_idx_load(b, b)
        pltpu.async_copy(pe_hbm, pe_v, pesem)
        for b in range(NBUF):
            wait_idx(b)
            start_gather(b, b)
        pltpu.make_async_copy(pe_hbm, pe_v, pesem).wait()

        def group_body(i, carry):
            for p in range(NBUF):
                b = p
                nb = (p + 2) % NBUF
                j = i * NBUF + p

                if p < 2:
                    @pl.when(i > 0)
                    def _():
                        wait_scatter(nb)
                        wait_idx(nb)
                        start_gather(j + 2, nb)
                else:
                    wait_scatter(nb)

                    @pl.when(i < n_groups - 1)
                    def _():
                        wait_idx(nb)
                        start_gather(j + 2, nb)

                wait_gather(b)
                @pl.when(i < n_groups - 1)
                def _():
                    start_idx_load(j + NBUF, b)

                compute(b)
                start_scatter(j, b)
            return carry

        lax.fori_loop(0, n_groups, group_body, 0)
        wait_scatter((n_chunks - 2) % NBUF)
        wait_scatter((n_chunks - 1) % NBUF)

    return k


def _pack_pe(pe):
    pe3 = pe.reshape(pe.shape[0], D // 32, 2, LANES)
    pb = jax.lax.bitcast_convert_type(
        pe3.astype(jnp.bfloat16), jnp.uint16
    ).astype(jnp.uint32)
    packed = pb[:, :, 0, :] | (pb[:, :, 1, :] << 16)
    return jax.lax.bitcast_convert_type(
        packed.reshape(pe.shape[0], D // 2), jnp.int32
    )


def kernel(x, table, pos_encoding):
    B, seq = x.shape
    N = B * seq
    info = plsc.get_sparse_core_info()
    nw = info.num_cores * info.num_subcores
    xr = x.reshape(nw, N // nw // G, G)
    pe = pos_encoding.reshape(pos_encoding.shape[1], pos_encoding.shape[2])[:seq]
    out = _make_kernel(N)(xr, _pack_pe(pe), table)
    return out.reshape(B, seq, D)


# speedup vs baseline: 1.0031x; 1.0031x over previous
import functools
import math

import jax
import jax.numpy as jnp
from jax import lax
from jax.experimental import pallas as pl
from jax.experimental.pallas import tpu as pltpu
from jax.experimental.pallas import tpu_sc as plsc

D = 128
S = 200
SCALE = math.sqrt(D)
LANES = 16
G = 100
NBUF = 4


def _make_kernel(N):
    info = plsc.get_sparse_core_info()
    NC, NS = info.num_cores, info.num_subcores
    NW = NC * NS
    n_per_w = N // NW
    n_chunks = n_per_w // S
    n_g = n_per_w // G
    n_groups = n_chunks // NBUF

    mesh = plsc.VectorSubcoreMesh(core_axis_name="c", subcore_axis_name="s")

    @functools.partial(
        pl.kernel,
        mesh=mesh,
        out_type=jax.ShapeDtypeStruct((N, D), jnp.float32),
        scratch_types=[
            pltpu.VMEM((S, D // 2), jnp.int32),
        ]
        + [pltpu.VMEM((2, G), jnp.int32) for _ in range(NBUF)]
        + [pltpu.VMEM((S, D), jnp.float32) for _ in range(NBUF)]
        + [pltpu.SemaphoreType.DMA for _ in range(3 * NBUF + 1)],
    )
    def k(x_hbm, pe_hbm, table_hbm, out_hbm, pe_v, *slabs_bufs_sems):
        idxs = slabs_bufs_sems[:NBUF]
        bufs = slabs_bufs_sems[NBUF:2 * NBUF]
        gsems = slabs_bufs_sems[2 * NBUF:3 * NBUF]
        ssems = slabs_bufs_sems[3 * NBUF:4 * NBUF]
        isems = slabs_bufs_sems[4 * NBUF:5 * NBUF]
        pesem = slabs_bufs_sems[5 * NBUF]

        wid = lax.axis_index("s") * NC + lax.axis_index("c")
        base_w = wid * n_per_w

        def start_idx_load(j, b):
            pltpu.async_copy(
                x_hbm.at[wid, pl.ds(2 * j, 2)], idxs[b], isems[b]
            )

        def wait_idx(b):
            pltpu.make_async_copy(
                x_hbm.at[wid, pl.ds(0, 2)], idxs[b], isems[b]
            ).wait()

        def start_gather(j, b):
            pltpu.async_copy(
                table_hbm.at[idxs[b].at[0]], bufs[b].at[pl.ds(0, G)], gsems[b]
            )
            pltpu.async_copy(
                table_hbm.at[idxs[b].at[1]], bufs[b].at[pl.ds(G, G)],
                gsems[b],
            )

        def wait_gather(b):
            pltpu.make_async_copy(
                table_hbm.at[pl.ds(0, S)], bufs[b], gsems[b]
            ).wait()

        def start_scatter(j, b):
            pltpu.async_copy(
                bufs[b], out_hbm.at[pl.ds(base_w + j * S, S)], ssems[b]
            )

        def wait_scatter(b):
            pltpu.make_async_copy(
                bufs[b], out_hbm.at[pl.ds(0, S)], ssems[b]
            ).wait()

        mask_hi = jnp.int32(-65536)

        def compute(b):
            buf = bufs[b]

            def do_row(r):
                for g in range(D // 32):
                    w = pe_v[r, pl.ds(g * LANES, LANES)]
                    lo = lax.bitcast_convert_type(
                        lax.shift_left(w, 16), jnp.float32)
                    hi = lax.bitcast_convert_type(
                        lax.bitwise_and(w, mask_hi), jnp.float32)
                    sl0 = pl.ds(g * 32, LANES)
                    sl1 = pl.ds(g * 32 + LANES, LANES)
                    buf[r, sl0] = buf[r, sl0] * SCALE + lo
                    buf[r, sl1] = buf[r, sl1] * SCALE + hi

            def row_body(r2, rc):
                do_row(2 * r2)
                do_row(2 * r2 + 1)
                return rc

            lax.fori_loop(0, S // 2, row_body, 0)

        for b in range(NBUF):
            start_idx_load(b, b)
        pltpu.async_copy(pe_hbm, pe_v, pesem)
        for b in range(NBUF):
            wait_idx(b)
            start_gather(b, b)
        pltpu.make_async_copy(pe_hbm, pe_v, pesem).wait()

        def group_body(i, carry):
            for p in range(NBUF):
                b = p
                nb = (p + 2) % NBUF
                j = i * NBUF + p

                if p < 2:
                    @pl.when(i > 0)
                    def _():
                        wait_scatter(nb)
                        wait_idx(nb)
                        start_gather(j + 2, nb)
                else:
                    wait_scatter(nb)

                    @pl.when(i < n_groups - 1)
                    def _():
                        wait_idx(nb)
                        start_gather(j + 2, nb)

                wait_gather(b)
                @pl.when(i < n_groups - 1)
                def _():
                    start_idx_load(j + NBUF, b)

                compute(b)
                start_scatter(j, b)
            return carry

        lax.fori_loop(0, n_groups, group_body, 0)
        wait_scatter((n_chunks - 2) % NBUF)
        wait_scatter((n_chunks - 1) % NBUF)

    return k


def _pack_pe(pe):
    pe3 = pe.reshape(pe.shape[0], D // 32, 2, LANES)
    pb = jax.lax.bitcast_convert_type(
        pe3.astype(jnp.bfloat16), jnp.uint16
    ).astype(jnp.uint32)
    packed = pb[:, :, 0, :] | (pb[:, :, 1, :] << 16)
    return jax.lax.bitcast_convert_type(
        packed.reshape(pe.shape[0], D // 2), jnp.int32
    )


def kernel(x, table, pos_encoding):
    B, seq = x.shape
    N = B * seq
    info = plsc.get_sparse_core_info()
    nw = info.num_cores * info.num_subcores
    xr = x.reshape(nw, N // nw // G, G)
    pe = pos_encoding.reshape(pos_encoding.shape[1], pos_encoding.shape[2])[:seq]
    out = _make_kernel(N)(xr, _pack_pe(pe), table)
    return out.reshape(B, seq, D)
